# Initial kernel scaffold; baseline (speedup 1.0000x reference)
import jax
import jax.numpy as jnp
from jax.experimental import pallas as pl


def kernel(txt, token_table, pos_table):
    raise NotImplementedError("write your pallas kernel here")


# speedup vs baseline: 1.1650x; 1.1650x over previous
import functools

import jax
import jax.numpy as jnp
from jax import lax
from jax.experimental import pallas as pl
from jax.experimental.pallas import tpu as pltpu
from jax.experimental.pallas import tpu_sc as plsc

_NC = 2
_NS = 16
_NW = _NC * _NS
_LANES = 16


def kernel(txt, token_table, pos_table):
    B, L = txt.shape
    V, D = token_table.shape
    assert B % _NW == 0 and D % _LANES == 0
    bpw = B // _NW
    C0 = min(40, L)
    C1 = L - C0
    DV = D // _LANES

    mesh = plsc.VectorSubcoreMesh(core_axis_name="c", subcore_axis_name="s")

    @functools.partial(
        pl.kernel,
        out_type=jax.ShapeDtypeStruct((B, L, D), jnp.float32),
        mesh=mesh,
        compiler_params=pltpu.CompilerParams(use_tc_tiling_on_sc=False),
        scratch_types=[
            pltpu.VMEM((L,), jnp.int32),
            pltpu.VMEM((C0, D), jnp.float32),
            pltpu.VMEM((C0, D), jnp.float32),
            pltpu.VMEM((L, D), jnp.float32),
            pltpu.SemaphoreType.DMA,
        ],
    )
    def run(txt_hbm, tok_hbm, pos_hbm, out_hbm, txt_v, buf_a, buf_b, pos_v, sem):
        wid = lax.axis_index("s") * _NC + lax.axis_index("c")
        b0 = wid * bpw
        pltpu.sync_copy(pos_hbm, pos_v)

        def do_chunk(b, l0, rows, buf):
            pltpu.async_copy(
                tok_hbm.at[txt_v.at[pl.ds(l0, rows)]],
                buf.at[pl.ds(0, rows)],
                sem,
            ).wait()

            def row_body(r, carry):
                for j in range(DV):
                    v = pos_v[l0 + r, pl.ds(_LANES * j, _LANES)]
                    plsc.addupdate(buf.at[r, pl.ds(_LANES * j, _LANES)], v)
                return carry

            lax.fori_loop(0, rows, row_body, 0)
            pltpu.sync_copy(buf.at[pl.ds(0, rows)], out_hbm.at[b, pl.ds(l0, rows)])

        def b_body(i, carry):
            b = b0 + i
            pltpu.sync_copy(txt_hbm.at[b], txt_v)
            do_chunk(b, 0, C0, buf_a)
            if C1:
                do_chunk(b, C0, C1, buf_b)
            return carry

        lax.fori_loop(0, bpw, b_body, 0)

    return run(txt, token_table, pos_table)


# speedup vs baseline: 1.4266x; 1.2245x over previous
import functools

import jax
import jax.numpy as jnp
from jax import lax
from jax.experimental import pallas as pl
from jax.experimental.pallas import tpu as pltpu
from jax.experimental.pallas import tpu_sc as plsc

_NC = 2
_NS = 16
_NW = _NC * _NS
_LANES = 16


def kernel(txt, token_table, pos_table):
    B, L = txt.shape
    V, D = token_table.shape
    assert B % (2 * _NW) == 0 and D % _LANES == 0
    bpw = B // _NW
    K = bpw // 2
    chunks = []
    l0 = 0
    while l0 < L:
        r = min(24, L - l0)
        chunks.append((l0, r))
        l0 += r
    cmax = max(r for _, r in chunks)
    DV = D // _LANES

    mesh = plsc.VectorSubcoreMesh(core_axis_name="c", subcore_axis_name="s")

    @functools.partial(
        pl.kernel,
        out_type=jax.ShapeDtypeStruct((B, L, D), jnp.float32),
        mesh=mesh,
        compiler_params=pltpu.CompilerParams(use_tc_tiling_on_sc=False),
        scratch_types=[
            pltpu.VMEM((L,), jnp.int32),
            pltpu.VMEM((L,), jnp.int32),
            [pltpu.VMEM((cmax, D), jnp.float32) for _ in chunks],
            pltpu.VMEM((L, D), jnp.float32),
            [pltpu.SemaphoreType.DMA for _ in chunks],
            [pltpu.SemaphoreType.DMA for _ in chunks],
            pltpu.SemaphoreType.DMA,
            pltpu.SemaphoreType.DMA,
        ],
    )
    def run(txt_hbm, tok_hbm, pos_hbm, out_hbm,
            txt_v0, txt_v1, bufs, pos_v, sems_in, sems_out, sem_t0, sem_t1):
        wid = lax.axis_index("s") * _NC + lax.axis_index("c")
        b0 = wid * bpw

        def gather(c, txt_ref):
            l0, r = chunks[c]
            return pltpu.async_copy(
                tok_hbm.at[txt_ref.at[pl.ds(l0, r)]],
                bufs[c].at[pl.ds(0, r)],
                sems_in[c],
            )

        def gather_wait(c, txt_ref):
            l0, r = chunks[c]
            pltpu.make_async_copy(
                tok_hbm.at[txt_ref.at[pl.ds(l0, r)]],
                bufs[c].at[pl.ds(0, r)],
                sems_in[c],
            ).wait()

        def scatter(c, b):
            l0, r = chunks[c]
            return pltpu.async_copy(
                bufs[c].at[pl.ds(0, r)],
                out_hbm.at[b, pl.ds(l0, r)],
                sems_out[c],
            )

        def add_pos(c):
            l0, r = chunks[c]

            def row_body(rr, carry):
                for j in range(DV):
                    v = pos_v[l0 + rr, pl.ds(_LANES * j, _LANES)]
                    plsc.addupdate(bufs[c].at[rr, pl.ds(_LANES * j, _LANES)], v)
                return carry

            lax.fori_loop(0, r, row_body, 0)

        pltpu.sync_copy(pos_hbm, pos_v)
        pltpu.sync_copy(txt_hbm.at[b0], txt_v0)
        pltpu.async_copy(txt_hbm.at[b0 + 1], txt_v1, sem_t1)
        for c in range(len(chunks)):
            gather(c, txt_v0)

        def body(k, carry):
            b = b0 + 2 * k
            not_last = k < K - 1

            @pl.when(not_last)
            def _():
                pltpu.async_copy(txt_hbm.at[b + 2], txt_v0, sem_t0)

            sc_a = []
            for c in range(len(chunks)):
                gather_wait(c, txt_v0)
                add_pos(c)
                sc_a.append(scatter(c, b))

            pltpu.make_async_copy(txt_hbm.at[b + 1], txt_v1, sem_t1).wait()
            g_b = []
            for c in range(len(chunks)):
                sc_a[c].wait()
                g_b.append(gather(c, txt_v1))

            @pl.when(not_last)
            def _():
                pltpu.make_async_copy(txt_hbm.at[b + 2], txt_v0, sem_t0).wait()

            sc_b = []
            for c in range(len(chunks)):
                g_b[c].wait()
                add_pos(c)
                sc_b.append(scatter(c, b + 1))

            @pl.when(not_last)
            def _():
                pltpu.async_copy(txt_hbm.at[b + 3], txt_v1, sem_t1)

            for c in range(len(chunks)):
                sc_b[c].wait()

                @pl.when(not_last)
                def _():
                    gather(c, txt_v0)

            return carry

        lax.fori_loop(0, K, body, 0)

    return run(txt, token_table, pos_table)


# speedup vs baseline: 4.4832x; 3.1427x over previous
import functools

import jax
import jax.numpy as jnp
from jax import lax
from jax.experimental import pallas as pl
from jax.experimental.pallas import tpu as pltpu
from jax.experimental.pallas import tpu_sc as plsc

_NC = 2
_NS = 16
_NW = _NC * _NS
_LANES = 16


def kernel(txt, token_table, pos_table):
    B, L = txt.shape
    V, D = token_table.shape
    DS = D // 128
    NP = B // 16
    PPW = NP // _NW
    assert D % 128 == 0 and B % (16 * _NW) == 0
    NIDX = 16 * DS
    NVEC = NIDX // _LANES
    UNROLL = 4
    assert PPW % UNROLL == 0
    K2 = L * PPW // UNROLL

    mesh = plsc.VectorSubcoreMesh(core_axis_name="c", subcore_axis_name="s")

    @functools.partial(
        pl.kernel,
        out_type=jax.ShapeDtypeStruct((L, NP, NIDX, 128), jnp.float32),
        mesh=mesh,
        compiler_params=pltpu.CompilerParams(
            use_tc_tiling_on_sc=False, needs_layout_passes=False),
        scratch_types=[
            pltpu.VMEM((16 * PPW,), jnp.int32),
            pltpu.VMEM((D,), jnp.float32),
            [pltpu.VMEM((NIDX,), jnp.int32) for _ in range(UNROLL)],
            [pltpu.VMEM((NIDX, 128), jnp.float32) for _ in range(UNROLL)],
            [pltpu.SemaphoreType.DMA for _ in range(UNROLL)],
            [pltpu.SemaphoreType.DMA for _ in range(UNROLL)],
        ],
    )
    def run(txtT_hbm, tokT_hbm, pos_hbm, out_hbm,
            txt_v, pos_v, idx_bufs, gbufs, sems_in, sems_out):
        wid = lax.axis_index("s") * _NC + lax.axis_index("c")

        def stage_l(l):
            pltpu.sync_copy(txtT_hbm.at[l, pl.ds(16 * PPW * wid, 16 * PPW)], txt_v)
            pltpu.sync_copy(pos_hbm.at[l], pos_v)

        def build_idx(q, j):
            iot = lax.iota(jnp.int32, _LANES)
            lo = iot & 7
            hi = iot >> 3
            for c in range(NVEC):
                g = 16 * j + 8 * (c // 3) + lo
                vals = plsc.load_gather(txt_v, [g])
                idx_bufs[q][pl.ds(16 * c, 16)] = vals * DS + (hi + 2 * (c % 3))

        def gather(q):
            return pltpu.async_copy(tokT_hbm.at[idx_bufs[q]], gbufs[q], sems_in[q])

        def gather_wait(q):
            pltpu.make_async_copy(tokT_hbm.at[idx_bufs[q]], gbufs[q], sems_in[q]).wait()

        def add_pos(q):
            def dt_body(dt, carry):
                for jj in range(8):
                    v = pos_v[pl.ds(dt * 128 + 16 * jj, 16)]
                    for t in range(2):
                        row = t * (8 * DS) + dt * 8
                        for br in range(8):
                            plsc.addupdate(
                                gbufs[q].at[row + br, pl.ds(16 * jj, 16)], v)
                return carry

            lax.fori_loop(0, DS, dt_body, 0)

        def scatter(q, l, pt):
            return pltpu.async_copy(gbufs[q], out_hbm.at[l, pt], sems_out[q])

        stage_l(0)
        for q in range(UNROLL):
            build_idx(q, q)
            gather(q)

        def body(k, carry):
            m = k % (PPW // UNROLL)
            l = k // (PPW // UNROLL)
            not_last = k < K2 - 1

            sc = []
            for q in range(UNROLL):
                gather_wait(q)
                add_pos(q)
                sc.append(scatter(q, l, PPW * wid + UNROLL * m + q))

            @pl.when((m == PPW // UNROLL - 1) & not_last)
            def _():
                stage_l(l + 1)

            for q in range(UNROLL):
                sc[q].wait()

                @pl.when(not_last)
                def _():
                    jn = (UNROLL * (m + 1) + q) % PPW
                    build_idx(q, jn)
                    gather(q)

            return carry

        lax.fori_loop(0, K2, body, 0)

    txtT = txt.T
    tokT = token_table.reshape(V * DS, 128)
    out5 = run(txtT, tokT, pos_table)
    return (out5.reshape(L, NP, 2, DS, 8, 128)
                .transpose(1, 2, 4, 0, 3, 5)
                .reshape(B, L, D))
